# baseline (speedup 1.0000x reference)
import functools

import jax
import jax.numpy as jnp
from jax.experimental import pallas as pl
from jax.experimental.pallas import tpu as pltpu

_EPS = 1e-6


def _vq_block_kernel(x_ref, xt_ref, cb_ref, x2_ref, x2r_ref, c2_ref,
                     cbaugt_ref, out_ref, idx_ref, loss_ref, *, scale):
    x = x_ref[...]
    xt = xt_ref[...]
    cb = cb_ref[...]
    x2 = x2_ref[...]
    x2r = x2r_ref[0]
    c2 = c2_ref[...]
    cbaugt = cbaugt_ref[...]
    d = x.shape[1]
    kk = cb.shape[0]
    b = x.shape[0]

    xm2 = x * (-2.0)
    xc2 = jax.lax.dot_general(
        xm2, cb, dimension_numbers=(((1,), (1,)), ((), ())),
        preferred_element_type=jnp.float32)
    d2c = jnp.maximum((x2 + c2) + xc2, 0.0)
    dist = d2c * jnp.minimum(jax.lax.rsqrt(d2c), 1e38)

    mn = jnp.min(dist, axis=1, keepdims=True)
    iota = jax.lax.broadcasted_iota(jnp.int32, dist.shape, 1)
    key = jnp.where(dist == mn, iota, jnp.int32(2**30))
    idx = jnp.min(key, axis=1)

    idx_ref[0, 0, :] = idx

    idxr = idx.reshape(1, b)
    kiota = jax.lax.broadcasted_iota(jnp.int32, (kk, b), 0)
    onehot_t = (kiota == idxr).astype(jnp.float32)
    qa_t = jax.lax.dot_general(
        cbaugt, onehot_t, dimension_numbers=(((1,), (0,)), ((), ())),
        preferred_element_type=jnp.float32)
    qt = qa_t[:d, :]
    nt2 = qa_t[d:d + 1, :]

    ones_d = jnp.ones((1, d), dtype=jnp.float32)
    xq = jax.lax.dot_general(
        ones_d, xt * qt, dimension_numbers=(((1,), (0,)), ((), ())),
        preferred_element_type=jnp.float32)

    nsc = jnp.maximum(jnp.sqrt(x2r), _EPS)
    ntc = jnp.maximum(jnp.sqrt(nt2), _EPS)
    invns = 1.0 / nsc
    invnt = 1.0 / ntc
    xu = x2r * invns
    t = xq * invnt
    nw2 = (xu + t * 2.0) * invns + nt2 * (invnt * invnt)
    invnw = 1.0 / jnp.maximum(jnp.sqrt(nw2), _EPS)
    dw = (xu + t) * (invnw * invnw)
    s = jnp.sqrt(nt2) * invns
    asc = (1.0 - 2.0 * dw * invns) * s
    bsc = (2.0 * invnt * (xu - dw)) * s
    out_ref[...] = (asc * xt + bsc * qt).T

    row = x2r - 2.0 * xq + nt2
    loss_ref[...] = (jnp.sum(row) * scale).reshape(1, 1, 1)


def kernel(x, codebook):
    n, d = x.shape
    k = codebook.shape[0]
    block = 2048
    nb = n // block

    x2 = jnp.sum(x * x, axis=-1, keepdims=True)
    c2 = jnp.sum(codebook * codebook, axis=-1)[None, :]
    xt = x.T
    cbaugt = jnp.concatenate(
        [codebook.T, c2, jnp.zeros((7, k), jnp.float32)], axis=0)

    out, idx3, loss_parts = pl.pallas_call(
        functools.partial(_vq_block_kernel, scale=1.25 / (n * d)),
        grid=(nb,),
        in_specs=[
            pl.BlockSpec((block, d), lambda i: (i, 0)),
            pl.BlockSpec((d, block), lambda i: (0, i)),
            pl.BlockSpec((k, d), lambda i: (0, 0)),
            pl.BlockSpec((block, 1), lambda i: (i, 0)),
            pl.BlockSpec((1, 1, block), lambda i: (i, 0, 0)),
            pl.BlockSpec((1, k), lambda i: (0, 0)),
            pl.BlockSpec((40, k), lambda i: (0, 0)),
        ],
        out_specs=[
            pl.BlockSpec((block, d), lambda i: (i, 0)),
            pl.BlockSpec((1, 1, block), lambda i: (i, 0, 0)),
            pl.BlockSpec((1, 1, 1), lambda i: (i, 0, 0)),
        ],
        out_shape=[
            jax.ShapeDtypeStruct((n, d), jnp.float32),
            jax.ShapeDtypeStruct((nb, 1, block), jnp.int32),
            jax.ShapeDtypeStruct((nb, 1, 1), jnp.float32),
        ],
        compiler_params=pltpu.CompilerParams(
            dimension_semantics=("parallel",)),
    )(x, xt, codebook, x2, x2.reshape(nb, 1, block), c2, cbaugt)

    return out, idx3.reshape(n), jnp.sum(loss_parts)


# speedup vs baseline: 1.1739x; 1.1739x over previous
import functools

import jax
import jax.numpy as jnp
from jax.experimental import pallas as pl
from jax.experimental.pallas import tpu as pltpu

_EPS = 1e-6


def _vq_block_kernel(x_ref, xt_ref, cb_ref, x2_ref, x2r_ref, c2_ref,
                     cbaugt_ref, out_ref, idx_ref, loss_ref, *, scale):
    x = x_ref[...]
    xt = xt_ref[...]
    cb = cb_ref[...]
    x2 = x2_ref[...]
    x2r = x2r_ref[0]
    c2 = c2_ref[...]
    cbaugt = cbaugt_ref[...]
    d = x.shape[1]
    kk = cb.shape[0]
    b = x.shape[0]

    xm2 = x * (-2.0)
    xc2 = jax.lax.dot_general(
        xm2, cb, dimension_numbers=(((1,), (1,)), ((), ())),
        preferred_element_type=jnp.float32)
    d2c = jnp.maximum((x2 + c2) + xc2, 0.0)
    dist = d2c * jnp.minimum(jax.lax.rsqrt(d2c), 1e38)

    mn = jnp.min(dist, axis=1, keepdims=True)
    iota = jax.lax.broadcasted_iota(jnp.int32, dist.shape, 1)
    key = jnp.where(dist == mn, iota, jnp.int32(2**30))
    idx = jnp.min(key, axis=1)

    idx_ref[0, 0, :] = idx

    idxr = idx.reshape(1, b)
    kiota = jax.lax.broadcasted_iota(jnp.int32, (kk, b), 0)
    onehot_t = (kiota == idxr).astype(jnp.float32)
    qa_t = jax.lax.dot_general(
        cbaugt, onehot_t, dimension_numbers=(((1,), (0,)), ((), ())),
        preferred_element_type=jnp.float32)
    qt = qa_t[:d, :]
    nt2 = qa_t[d:d + 1, :]

    ones_d = jnp.ones((1, d), dtype=jnp.float32)
    xq = jax.lax.dot_general(
        ones_d, xt * qt, dimension_numbers=(((1,), (0,)), ((), ())),
        preferred_element_type=jnp.float32)

    nsc = jnp.maximum(jnp.sqrt(x2r), _EPS)
    ntc = jnp.maximum(jnp.sqrt(nt2), _EPS)
    invns = 1.0 / nsc
    invnt = 1.0 / ntc
    xu = x2r * invns
    t = xq * invnt
    nw2 = (xu + t * 2.0) * invns + nt2 * (invnt * invnt)
    invnw = 1.0 / jnp.maximum(jnp.sqrt(nw2), _EPS)
    dw = (xu + t) * (invnw * invnw)
    s = jnp.sqrt(nt2) * invns
    asc = (1.0 - 2.0 * dw * invns) * s
    bsc = (2.0 * invnt * (xu - dw)) * s
    out_ref[...] = asc * xt + bsc * qt

    row = x2r - 2.0 * xq + nt2
    loss_ref[...] = (jnp.sum(row) * scale).reshape(1, 1, 1)


def kernel(x, codebook):
    n, d = x.shape
    k = codebook.shape[0]
    block = 2048
    nb = n // block

    x2 = jnp.sum(x * x, axis=-1, keepdims=True)
    c2 = jnp.sum(codebook * codebook, axis=-1)[None, :]
    xt = x.T
    cbaugt = jnp.concatenate(
        [codebook.T, c2, jnp.zeros((7, k), jnp.float32)], axis=0)

    out_t, idx3, loss_parts = pl.pallas_call(
        functools.partial(_vq_block_kernel, scale=1.25 / (n * d)),
        grid=(nb,),
        in_specs=[
            pl.BlockSpec((block, d), lambda i: (i, 0)),
            pl.BlockSpec((d, block), lambda i: (0, i)),
            pl.BlockSpec((k, d), lambda i: (0, 0)),
            pl.BlockSpec((block, 1), lambda i: (i, 0)),
            pl.BlockSpec((1, 1, block), lambda i: (i, 0, 0)),
            pl.BlockSpec((1, k), lambda i: (0, 0)),
            pl.BlockSpec((40, k), lambda i: (0, 0)),
        ],
        out_specs=[
            pl.BlockSpec((d, block), lambda i: (0, i)),
            pl.BlockSpec((1, 1, block), lambda i: (i, 0, 0)),
            pl.BlockSpec((1, 1, 1), lambda i: (i, 0, 0)),
        ],
        out_shape=[
            jax.ShapeDtypeStruct((d, n), jnp.float32),
            jax.ShapeDtypeStruct((nb, 1, block), jnp.int32),
            jax.ShapeDtypeStruct((nb, 1, 1), jnp.float32),
        ],
        compiler_params=pltpu.CompilerParams(
            dimension_semantics=("parallel",)),
    )(x, xt, codebook, x2, x2.reshape(nb, 1, block), c2, cbaugt)

    return out_t.T, idx3.reshape(n), jnp.sum(loss_parts)


# speedup vs baseline: 1.1912x; 1.0147x over previous
import functools

import jax
import jax.numpy as jnp
from jax.experimental import pallas as pl
from jax.experimental.pallas import tpu as pltpu

_EPS = 1e-6


def _vq_block_kernel(x_ref, xt_ref, cb_ref, x2_ref, x2r_ref, c2_ref,
                     cbaugt_ref, out_ref, idx_ref, loss_ref, *, scale):
    x = x_ref[...]
    xt = xt_ref[...]
    cb = cb_ref[...]
    x2 = x2_ref[...]
    x2r = x2r_ref[0]
    c2 = c2_ref[...]
    cbaugt = cbaugt_ref[...]
    d = x.shape[1]
    kk = cb.shape[0]
    b = x.shape[0]

    xm2 = x * (-2.0)
    xc2 = jax.lax.dot_general(
        xm2, cb, dimension_numbers=(((1,), (1,)), ((), ())),
        preferred_element_type=jnp.float32)
    d2c = jnp.maximum((x2 + c2) + xc2, 0.0)
    dist = d2c * jnp.minimum(jax.lax.rsqrt(d2c), 1e38)

    mn = jnp.min(dist, axis=1, keepdims=True)
    iota = jax.lax.broadcasted_iota(jnp.int32, dist.shape, 1)
    key = jnp.where(dist == mn, iota, jnp.int32(2**30))
    idx = jnp.min(key, axis=1)

    idx_ref[0, 0, :] = idx

    idxr = idx.reshape(1, b)
    kiota = jax.lax.broadcasted_iota(jnp.int32, (kk, b), 0)
    onehot_t = (kiota == idxr).astype(jnp.float32)
    qa_t = jax.lax.dot_general(
        cbaugt, onehot_t, dimension_numbers=(((1,), (0,)), ((), ())),
        preferred_element_type=jnp.float32)
    qt = qa_t[:d, :]
    nt2 = qa_t[d:d + 1, :]

    ones_d = jnp.ones((1, d), dtype=jnp.float32)
    xq = jax.lax.dot_general(
        ones_d, xt * qt, dimension_numbers=(((1,), (0,)), ((), ())),
        preferred_element_type=jnp.float32)

    nsc = jnp.maximum(jnp.sqrt(x2r), _EPS)
    ntc = jnp.maximum(jnp.sqrt(nt2), _EPS)
    invns = 1.0 / nsc
    invnt = 1.0 / ntc
    xu = x2r * invns
    t = xq * invnt
    nw2 = (xu + t * 2.0) * invns + nt2 * (invnt * invnt)
    invnw = 1.0 / jnp.maximum(jnp.sqrt(nw2), _EPS)
    dw = (xu + t) * (invnw * invnw)
    s = jnp.sqrt(nt2) * invns
    asc = (1.0 - 2.0 * dw * invns) * s
    bsc = (2.0 * invnt * (xu - dw)) * s
    out_ref[...] = asc * xt + bsc * qt

    row = x2r - 2.0 * xq + nt2
    loss_ref[...] = (jnp.sum(row) * scale).reshape(1, 1, 1)


def kernel(x, codebook):
    n, d = x.shape
    k = codebook.shape[0]
    block = 4096
    nb = n // block

    x2 = jnp.sum(x * x, axis=-1, keepdims=True)
    c2 = jnp.sum(codebook * codebook, axis=-1)[None, :]
    xt = x.T
    cbaugt = jnp.concatenate(
        [codebook.T, c2, jnp.zeros((7, k), jnp.float32)], axis=0)

    out_t, idx3, loss_parts = pl.pallas_call(
        functools.partial(_vq_block_kernel, scale=1.25 / (n * d)),
        grid=(nb,),
        in_specs=[
            pl.BlockSpec((block, d), lambda i: (i, 0)),
            pl.BlockSpec((d, block), lambda i: (0, i)),
            pl.BlockSpec((k, d), lambda i: (0, 0)),
            pl.BlockSpec((block, 1), lambda i: (i, 0)),
            pl.BlockSpec((1, 1, block), lambda i: (i, 0, 0)),
            pl.BlockSpec((1, k), lambda i: (0, 0)),
            pl.BlockSpec((40, k), lambda i: (0, 0)),
        ],
        out_specs=[
            pl.BlockSpec((d, block), lambda i: (0, i)),
            pl.BlockSpec((1, 1, block), lambda i: (i, 0, 0)),
            pl.BlockSpec((1, 1, 1), lambda i: (i, 0, 0)),
        ],
        out_shape=[
            jax.ShapeDtypeStruct((d, n), jnp.float32),
            jax.ShapeDtypeStruct((nb, 1, block), jnp.int32),
            jax.ShapeDtypeStruct((nb, 1, 1), jnp.float32),
        ],
        compiler_params=pltpu.CompilerParams(
            dimension_semantics=("parallel",)),
    )(x, xt, codebook, x2, x2.reshape(nb, 1, block), c2, cbaugt)

    return out_t.T, idx3.reshape(n), jnp.sum(loss_parts)
